# baseline (speedup 1.0000x reference)
import functools

import jax
import jax.numpy as jnp
from jax import lax
from jax.experimental import pallas as pl
from jax.experimental.pallas import tpu as pltpu
from jax.experimental.pallas import tpu_sc as plsc

NNODES = 10000
D = 128
NC = 2
NS = 16
NTILES = NC * NS
CH = 128
NPAD = 10112
RPT = NPAD // NS
RPB = 2528
NBLK = NPAD // RPB


def _sc_mesh():
    return plsc.VectorSubcoreMesh(core_axis_name="c", subcore_axis_name="s")


def _row_chunks(total, step):
    out = []
    q0 = 0
    while q0 < total:
        out.append((q0, min(step, total - q0)))
        q0 += step
    return out


def _chunk_bounds(nch):
    return [8 * (nch * w // (NTILES * 8)) for w in range(NTILES + 1)]


@functools.partial(jax.jit, static_argnums=(1, 2))
def _deg_call(dst_flat, nch, max_n):
    @functools.partial(
        pl.kernel,
        out_type=jax.ShapeDtypeStruct((NC * NPAD,), jnp.float32),
        mesh=_sc_mesh(),
        scratch_types=[
            pltpu.VMEM((max_n, CH), jnp.int32),
            pltpu.VMEM((CH,), jnp.float32),
            pltpu.VMEM((RPT,), jnp.float32),
            pltpu.VMEM_SHARED((NPAD,), jnp.float32),
            pltpu.SemaphoreType.DMA,
        ],
    )
    def deg_kernel(dst_hbm, zrow_hbm, ones_hbm, out_hbm, didx, ones_v,
                   stage_v, acc_sh, dsem):
        c = lax.axis_index("c")
        s = lax.axis_index("s")
        w = c * NS + s
        rw = 8 * ((nch * w) // (NTILES * 8))
        rw1 = 8 * ((nch * (w + 1)) // (NTILES * 8))
        n_w = rw1 - rw
        pltpu.sync_copy(ones_hbm, ones_v)
        pltpu.sync_copy(zrow_hbm, stage_v)
        pltpu.sync_copy(stage_v, acc_sh.at[pl.ds(s * RPT, RPT)])
        pltpu.sync_copy(dst_hbm.at[pl.ds(rw, max_n)], didx)
        plsc.subcore_barrier()

        def body(j, carry):
            pltpu.async_copy(ones_v, acc_sh.at[didx.at[j]], dsem, add=True)
            return carry

        lax.fori_loop(0, n_w, body, 0)

        def drain(j, carry):
            pltpu.make_async_copy(ones_v, acc_sh.at[didx.at[j]], dsem).wait()
            return carry

        lax.fori_loop(0, n_w, drain, 0)
        plsc.subcore_barrier()
        pltpu.sync_copy(acc_sh.at[pl.ds(s * RPT, RPT)], stage_v)
        pltpu.sync_copy(stage_v, out_hbm.at[pl.ds(c * NPAD + s * RPT, RPT)])

    zrow = jnp.zeros((RPT,), jnp.float32)
    ones = jnp.ones((CH,), jnp.float32)
    return deg_kernel(dst_flat.reshape(-1, CH), zrow, ones)


@functools.partial(jax.jit, static_argnums=(3,))
def _agg_call(hp, src_flat, dst_flat, nch):
    wb_chunks = _row_chunks(RPT, CH)

    @functools.partial(
        pl.kernel,
        out_type=jax.ShapeDtypeStruct((NC * NPAD, D), jnp.float32),
        mesh=_sc_mesh(),
        scratch_types=[
            pltpu.VMEM((3, CH), jnp.int32),
            pltpu.VMEM((3, CH), jnp.int32),
            pltpu.VMEM((3, CH, D), jnp.float32),
            pltpu.VMEM_SHARED((NPAD, D), jnp.float32),
        ] + [pltpu.SemaphoreType.DMA] * 12,
    )
    def agg_kernel(hp_hbm, src_hbm, dst_hbm, zrows_hbm, out_hbm,
                   sidx, didx, rows, acc_sh,
                   g0, g1, g2, t0, t1, t2, i0, i1, i2, d0, d1, d2):
        gs = (g0, g1, g2)
        ts = (t0, t1, t2)
        js = (i0, i1, i2)
        ds_ = (d0, d1, d2)
        c = lax.axis_index("c")
        s = lax.axis_index("s")
        w = c * NS + s
        r0 = s * RPT
        rw = 8 * ((nch * w) // (NTILES * 8))
        rw1 = 8 * ((nch * (w + 1)) // (NTILES * 8))
        n_w = rw1 - rw

        pltpu.sync_copy(zrows_hbm, rows.at[0])
        for q0, qn in wb_chunks:
            pltpu.sync_copy(rows.at[0, pl.ds(0, qn)],
                            acc_sh.at[pl.ds(r0 + q0, qn), :])
        plsc.subcore_barrier()

        def sidx_cp(j, b):
            off = pl.multiple_of((rw + j) * CH, CH)
            return pltpu.make_async_copy(src_hbm.at[pl.ds(off, CH)],
                                         sidx.at[b], js[b])

        def didx_cp(j, b):
            off = pl.multiple_of((rw + j) * CH, CH)
            return pltpu.make_async_copy(dst_hbm.at[pl.ds(off, CH)],
                                         didx.at[b], ds_[b])

        def gather_cp(b):
            return pltpu.make_async_copy(hp_hbm.at[sidx.at[b]], rows.at[b],
                                         gs[b])

        def scat_start(b):
            pltpu.async_copy(rows.at[b], acc_sh.at[didx.at[b]], ts[b],
                             add=True)

        def scat_wait(b):
            pltpu.make_async_copy(rows.at[b], acc_sh.at[didx.at[b]],
                                  ts[b]).wait()

        sidx_cp(0, 0).start()
        didx_cp(0, 0).start()
        sidx_cp(1, 1).start()
        didx_cp(1, 1).start()
        sidx_cp(0, 0).wait()
        didx_cp(0, 0).wait()
        gather_cp(0).start()

        def body(jj, carry):
            for b in (0, 1, 2):
                j = jj * 3 + b
                b1 = (b + 1) % 3
                b2 = (b + 2) % 3

                @pl.when(j < n_w)
                def _process():
                    gather_cp(b).wait()
                    scat_start(b)

                    @pl.when(j + 2 < n_w)
                    def _prefetch_idx():
                        @pl.when(j >= 1)
                        def _w():
                            scat_wait(b2)
                        sidx_cp(j + 2, b2).start()
                        didx_cp(j + 2, b2).start()

                    @pl.when(j + 1 < n_w)
                    def _gather_next():
                        sidx_cp(j + 1, b1).wait()
                        didx_cp(j + 1, b1).wait()
                        gather_cp(b1).start()
            return carry

        lax.fori_loop(0, (n_w + 2) // 3, body, 0)
        for b in (0, 1, 2):
            scat_wait(b)
        plsc.subcore_barrier()

        def wb(i, phase):
            q0, qn = wb_chunks[i]
            b = i % 2
            cp = pltpu.make_async_copy(
                rows.at[b, pl.ds(0, qn)],
                out_hbm.at[pl.ds(c * NPAD + r0 + q0, qn), :], gs[b])
            if phase == 0:
                pltpu.sync_copy(acc_sh.at[pl.ds(r0 + q0, qn), :],
                                rows.at[b, pl.ds(0, qn)])
                cp.start()
            else:
                cp.wait()

        for i in range(len(wb_chunks)):
            if i >= 2:
                wb(i - 2, 1)
            wb(i, 0)
        for i in range(max(0, len(wb_chunks) - 2), len(wb_chunks)):
            wb(i, 1)

    zrows = jnp.zeros((CH, D), jnp.float32)
    return agg_kernel(hp, src_flat, dst_flat, zrows)


def _tc1_body(x_ref, w_ref, d0_ref, d1_ref, out_ref, dinv_ref):
    dinv = lax.rsqrt(d0_ref[...] + d1_ref[...] + 1.0)
    dinv_ref[...] = dinv
    h = jnp.dot(x_ref[...], w_ref[...], preferred_element_type=jnp.float32)
    out_ref[...] = h * dinv


def _tc1(x, w1, deg_col):
    return pl.pallas_call(
        _tc1_body,
        grid=(NBLK,),
        in_specs=[
            pl.BlockSpec((RPB, D), lambda i: (i, 0)),
            pl.BlockSpec((D, D), lambda i: (0, 0)),
            pl.BlockSpec((RPB, 1), lambda i: (i, 0)),
            pl.BlockSpec((RPB, 1), lambda i: (i + NBLK, 0)),
        ],
        out_specs=[
            pl.BlockSpec((RPB, D), lambda i: (i, 0)),
            pl.BlockSpec((RPB, 1), lambda i: (i, 0)),
        ],
        out_shape=[
            jax.ShapeDtypeStruct((NPAD, D), jnp.float32),
            jax.ShapeDtypeStruct((NPAD, 1), jnp.float32),
        ],
    )(x, w1, deg_col, deg_col)


def _tc2_body(a0_ref, a1_ref, hp_ref, dinv_ref, b_ref, w_ref, out_ref):
    pre = dinv_ref[...] * (a0_ref[...] + a1_ref[...] + hp_ref[...]) + b_ref[...]
    x2 = jnp.maximum(pre, 0.0)
    h = jnp.dot(x2, w_ref[...], preferred_element_type=jnp.float32)
    out_ref[...] = h * dinv_ref[...]


def _tc2(g1, h1p, dinv_col, b1r, w2):
    return pl.pallas_call(
        _tc2_body,
        grid=(NBLK,),
        in_specs=[
            pl.BlockSpec((RPB, D), lambda i: (i, 0)),
            pl.BlockSpec((RPB, D), lambda i: (i + NBLK, 0)),
            pl.BlockSpec((RPB, D), lambda i: (i, 0)),
            pl.BlockSpec((RPB, 1), lambda i: (i, 0)),
            pl.BlockSpec((1, D), lambda i: (0, 0)),
            pl.BlockSpec((D, D), lambda i: (0, 0)),
        ],
        out_specs=pl.BlockSpec((RPB, D), lambda i: (i, 0)),
        out_shape=jax.ShapeDtypeStruct((NPAD, D), jnp.float32),
    )(g1, g1, h1p, dinv_col, b1r, w2)


def _tc3_body(a0_ref, a1_ref, hp_ref, dinv_ref, b_ref, out_ref):
    pre = dinv_ref[...] * (a0_ref[...] + a1_ref[...] + hp_ref[...]) + b_ref[...]
    out_ref[...] = jnp.maximum(pre, 0.0)


def _tc3(g2, h2p, dinv_col, b2r):
    return pl.pallas_call(
        _tc3_body,
        grid=(NBLK,),
        in_specs=[
            pl.BlockSpec((RPB, D), lambda i: (i, 0)),
            pl.BlockSpec((RPB, D), lambda i: (i + NBLK, 0)),
            pl.BlockSpec((RPB, D), lambda i: (i, 0)),
            pl.BlockSpec((RPB, 1), lambda i: (i, 0)),
            pl.BlockSpec((1, D), lambda i: (0, 0)),
        ],
        out_specs=pl.BlockSpec((RPB, D), lambda i: (i, 0)),
        out_shape=jax.ShapeDtypeStruct((NNODES, D), jnp.float32),
    )(g2, g2, h2p, dinv_col, b2r)


def kernel(x, edge_index, W1, b1, W2, b2):
    e = edge_index.shape[1]
    nch = -(-e // CH)
    nch = -(-nch // 8) * 8
    bounds = _chunk_bounds(nch)
    max_n = max(b1_ - b0_ for b0_, b1_ in zip(bounds[:-1], bounds[1:]))
    tot = (nch + max_n) * CH
    pad = tot - e
    src = edge_index[0]
    dst = edge_index[1]
    if pad:
        ar = jnp.arange(pad, dtype=jnp.int32)
        src = jnp.concatenate([src, ar % NNODES])
        dst = jnp.concatenate([dst, NNODES + ar % (NPAD - NNODES)])

    deg2 = _deg_call(dst, nch, max_n)
    deg_col = deg2.reshape(NC * NPAD, 1)

    b1r = b1.reshape(1, D)
    b2r = b2.reshape(1, D)

    h1p, dinv_col = _tc1(x, W1, deg_col)
    g1 = _agg_call(h1p, src, dst, nch)
    h2p = _tc2(g1, h1p, dinv_col, b1r, W2)
    g2 = _agg_call(h2p, src, dst, nch)
    return _tc3(g2, h2p, dinv_col, b2r)


# speedup vs baseline: 1.2226x; 1.2226x over previous
import functools

import jax
import jax.numpy as jnp
from jax import lax
from jax.experimental import pallas as pl
from jax.experimental.pallas import tpu as pltpu
from jax.experimental.pallas import tpu_sc as plsc

NNODES = 10000
D = 128
NC = 2
NS = 16
NTILES = NC * NS
CH = 80
CHD = 128
NPAD = 10112
RPT = NPAD // NS
RPB = 2528
NBLK = NPAD // RPB


def _sc_mesh():
    return plsc.VectorSubcoreMesh(core_axis_name="c", subcore_axis_name="s")


def _row_chunks(total, step):
    out = []
    q0 = 0
    while q0 < total:
        out.append((q0, min(step, total - q0)))
        q0 += step
    return out


def _chunk_bounds(nch):
    return [8 * (nch * w // (NTILES * 8)) for w in range(NTILES + 1)]


def _part(nch):
    b = _chunk_bounds(nch)
    return max(y - x for x, y in zip(b[:-1], b[1:])), b[-2]


@functools.partial(jax.jit, static_argnums=(1, 2))
def _deg_call(dst_flat, nch, max_n):
    @functools.partial(
        pl.kernel,
        out_type=jax.ShapeDtypeStruct((NC * NPAD,), jnp.float32),
        mesh=_sc_mesh(),
        scratch_types=[
            pltpu.VMEM((max_n, CHD), jnp.int32),
            pltpu.VMEM((CHD,), jnp.float32),
            pltpu.VMEM((RPT,), jnp.float32),
            pltpu.VMEM_SHARED((NPAD,), jnp.float32),
            pltpu.SemaphoreType.DMA,
        ],
    )
    def deg_kernel(dst_hbm, zrow_hbm, ones_hbm, out_hbm, didx, ones_v,
                   stage_v, acc_sh, dsem):
        c = lax.axis_index("c")
        s = lax.axis_index("s")
        w = c * NS + s
        rw = 8 * ((nch * w) // (NTILES * 8))
        rw1 = 8 * ((nch * (w + 1)) // (NTILES * 8))
        n_w = rw1 - rw
        pltpu.sync_copy(ones_hbm, ones_v)
        pltpu.sync_copy(zrow_hbm, stage_v)
        pltpu.sync_copy(stage_v, acc_sh.at[pl.ds(s * RPT, RPT)])
        pltpu.sync_copy(dst_hbm.at[pl.ds(rw, max_n)], didx)
        plsc.subcore_barrier()

        def body(j, carry):
            pltpu.async_copy(ones_v, acc_sh.at[didx.at[j]], dsem, add=True)
            return carry

        lax.fori_loop(0, n_w, body, 0)

        def drain(j, carry):
            pltpu.make_async_copy(ones_v, acc_sh.at[didx.at[j]], dsem).wait()
            return carry

        lax.fori_loop(0, n_w, drain, 0)
        plsc.subcore_barrier()
        pltpu.sync_copy(acc_sh.at[pl.ds(s * RPT, RPT)], stage_v)
        pltpu.sync_copy(stage_v, out_hbm.at[pl.ds(c * NPAD + s * RPT, RPT)])

    zrow = jnp.zeros((RPT,), jnp.float32)
    ones = jnp.ones((CHD,), jnp.float32)
    return deg_kernel(dst_flat.reshape(-1, CHD), zrow, ones)


@functools.partial(jax.jit, static_argnums=(3, 4))
def _agg_call(hp, src_flat, dst_flat, nch, max_n):
    wb_chunks = _row_chunks(RPT, CH)

    @functools.partial(
        pl.kernel,
        out_type=jax.ShapeDtypeStruct((NC * NPAD, D), jnp.float32),
        mesh=_sc_mesh(),
        scratch_types=[
            pltpu.VMEM((max_n * CH,), jnp.int32),
            pltpu.VMEM((3, CH), jnp.int32),
            pltpu.VMEM((3, CH, D), jnp.float32),
            pltpu.VMEM_SHARED((NPAD, D), jnp.float32),
        ] + [pltpu.SemaphoreType.DMA] * 9,
    )
    def agg_kernel(hp_hbm, src_hbm, dst_hbm, zrows_hbm, out_hbm,
                   sidx, didx, rows, acc_sh,
                   g0, g1, g2, t0, t1, t2, d0, d1, d2):
        gs = (g0, g1, g2)
        ts = (t0, t1, t2)
        ds_ = (d0, d1, d2)
        c = lax.axis_index("c")
        s = lax.axis_index("s")
        w = c * NS + s
        r0 = s * RPT
        rw = 8 * ((nch * w) // (NTILES * 8))
        rw1 = 8 * ((nch * (w + 1)) // (NTILES * 8))
        n_w = rw1 - rw

        pltpu.sync_copy(src_hbm.at[pl.ds(rw * CH, max_n * CH)], sidx)

        pltpu.sync_copy(zrows_hbm, rows.at[0])
        for q0, qn in wb_chunks:
            pltpu.sync_copy(rows.at[0, pl.ds(0, qn)],
                            acc_sh.at[pl.ds(r0 + q0, qn), :])
        plsc.subcore_barrier()

        def didx_cp(j, b):
            off = pl.multiple_of((rw + j) * CH, CH)
            return pltpu.make_async_copy(dst_hbm.at[pl.ds(off, CH)],
                                         didx.at[b], ds_[b])

        def gather_cp(j, b):
            return pltpu.make_async_copy(
                hp_hbm.at[sidx.at[pl.ds(j * CH, CH)]], rows.at[b], gs[b])

        def scat_start(b):
            pltpu.async_copy(rows.at[b], acc_sh.at[didx.at[b]], ts[b],
                             add=True)

        def scat_wait(b):
            pltpu.make_async_copy(rows.at[b], acc_sh.at[didx.at[b]],
                                  ts[b]).wait()

        didx_cp(0, 0).start()
        didx_cp(1, 1).start()
        gather_cp(0, 0).start()
        gather_cp(1, 1).start()

        def body(jj, carry):
            for b in (0, 1, 2):
                j = jj * 3 + b
                b2 = (b + 2) % 3

                @pl.when(j < n_w)
                def _process():
                    gather_cp(j, b).wait()
                    didx_cp(j, b).wait()
                    scat_start(b)

                    @pl.when(j + 2 < n_w)
                    def _prefetch():
                        @pl.when(j >= 1)
                        def _w():
                            scat_wait(b2)
                        didx_cp(j + 2, b2).start()
                        gather_cp(j + 2, b2).start()
            return carry

        lax.fori_loop(0, (n_w + 2) // 3, body, 0)
        for b in (0, 1, 2):
            scat_wait(b)
        plsc.subcore_barrier()

        def wb(i, phase):
            q0, qn = wb_chunks[i]
            b = i % 2
            cp = pltpu.make_async_copy(
                rows.at[b, pl.ds(0, qn)],
                out_hbm.at[pl.ds(c * NPAD + r0 + q0, qn), :], gs[b])
            if phase == 0:
                pltpu.sync_copy(acc_sh.at[pl.ds(r0 + q0, qn), :],
                                rows.at[b, pl.ds(0, qn)])
                cp.start()
            else:
                cp.wait()

        for i in range(len(wb_chunks)):
            if i >= 2:
                wb(i - 2, 1)
            wb(i, 0)
        for i in range(max(0, len(wb_chunks) - 2), len(wb_chunks)):
            wb(i, 1)

    zrows = jnp.zeros((CH, D), jnp.float32)
    return agg_kernel(hp, src_flat, dst_flat, zrows)


def _tc1_body(x_ref, w_ref, d0_ref, d1_ref, out_ref, dinv_ref):
    dinv = lax.rsqrt(d0_ref[...] + d1_ref[...] + 1.0)
    dinv_ref[...] = dinv
    h = jnp.dot(x_ref[...], w_ref[...], preferred_element_type=jnp.float32)
    out_ref[...] = h * dinv


def _tc1(x, w1, deg_col):
    return pl.pallas_call(
        _tc1_body,
        grid=(NBLK,),
        in_specs=[
            pl.BlockSpec((RPB, D), lambda i: (i, 0)),
            pl.BlockSpec((D, D), lambda i: (0, 0)),
            pl.BlockSpec((RPB, 1), lambda i: (i, 0)),
            pl.BlockSpec((RPB, 1), lambda i: (i + NBLK, 0)),
        ],
        out_specs=[
            pl.BlockSpec((RPB, D), lambda i: (i, 0)),
            pl.BlockSpec((RPB, 1), lambda i: (i, 0)),
        ],
        out_shape=[
            jax.ShapeDtypeStruct((NPAD, D), jnp.float32),
            jax.ShapeDtypeStruct((NPAD, 1), jnp.float32),
        ],
    )(x, w1, deg_col, deg_col)


def _tc2_body(a0_ref, a1_ref, hp_ref, dinv_ref, b_ref, w_ref, out_ref):
    pre = dinv_ref[...] * (a0_ref[...] + a1_ref[...] + hp_ref[...]) + b_ref[...]
    x2 = jnp.maximum(pre, 0.0)
    h = jnp.dot(x2, w_ref[...], preferred_element_type=jnp.float32)
    out_ref[...] = h * dinv_ref[...]


def _tc2(g1, h1p, dinv_col, b1r, w2):
    return pl.pallas_call(
        _tc2_body,
        grid=(NBLK,),
        in_specs=[
            pl.BlockSpec((RPB, D), lambda i: (i, 0)),
            pl.BlockSpec((RPB, D), lambda i: (i + NBLK, 0)),
            pl.BlockSpec((RPB, D), lambda i: (i, 0)),
            pl.BlockSpec((RPB, 1), lambda i: (i, 0)),
            pl.BlockSpec((1, D), lambda i: (0, 0)),
            pl.BlockSpec((D, D), lambda i: (0, 0)),
        ],
        out_specs=pl.BlockSpec((RPB, D), lambda i: (i, 0)),
        out_shape=jax.ShapeDtypeStruct((NPAD, D), jnp.float32),
    )(g1, g1, h1p, dinv_col, b1r, w2)


def _tc3_body(a0_ref, a1_ref, hp_ref, dinv_ref, b_ref, out_ref):
    pre = dinv_ref[...] * (a0_ref[...] + a1_ref[...] + hp_ref[...]) + b_ref[...]
    out_ref[...] = jnp.maximum(pre, 0.0)


def _tc3(g2, h2p, dinv_col, b2r):
    return pl.pallas_call(
        _tc3_body,
        grid=(NBLK,),
        in_specs=[
            pl.BlockSpec((RPB, D), lambda i: (i, 0)),
            pl.BlockSpec((RPB, D), lambda i: (i + NBLK, 0)),
            pl.BlockSpec((RPB, D), lambda i: (i, 0)),
            pl.BlockSpec((RPB, 1), lambda i: (i, 0)),
            pl.BlockSpec((1, D), lambda i: (0, 0)),
        ],
        out_specs=pl.BlockSpec((RPB, D), lambda i: (i, 0)),
        out_shape=jax.ShapeDtypeStruct((NNODES, D), jnp.float32),
    )(g2, g2, h2p, dinv_col, b2r)


def kernel(x, edge_index, W1, b1, W2, b2):
    e = edge_index.shape[1]
    ncha = -(-(-(-e // CH)) // 8) * 8
    nchd = -(-(-(-e // CHD)) // 8) * 8
    max_na, lasta = _part(ncha)
    max_nd, lastd = _part(nchd)
    tot = max(ncha * CH, nchd * CHD,
              (lasta + max_na) * CH, (lastd + max_nd) * CHD)
    tot = -(-tot // (CH * CHD // 16)) * (CH * CHD // 16)
    pad = tot - e
    src = edge_index[0]
    dst = edge_index[1]
    if pad:
        ar = jnp.arange(pad, dtype=jnp.int32)
        src = jnp.concatenate([src, ar % NNODES])
        dst = jnp.concatenate([dst, NNODES + ar % (NPAD - NNODES)])

    deg2 = _deg_call(dst, nchd, max_nd)
    deg_col = deg2.reshape(NC * NPAD, 1)

    b1r = b1.reshape(1, D)
    b2r = b2.reshape(1, D)

    h1p, dinv_col = _tc1(x, W1, deg_col)
    g1 = _agg_call(h1p, src, dst, ncha, max_na)
    h2p = _tc2(g1, h1p, dinv_col, b1r, W2)
    g2 = _agg_call(h2p, src, dst, ncha, max_na)
    return _tc3(g2, h2p, dinv_col, b2r)
